# baseline (speedup 1.0000x reference)
import functools

import jax
import jax.numpy as jnp
from jax import lax
from jax.experimental import pallas as pl
from jax.experimental.pallas import tpu as pltpu
from jax.experimental.pallas import tpu_sc as plsc

_NC = 2
_NS = 16
_NW = _NC * _NS
_CH = 64
_NBUF = 6
_SLAB = 632


def _make_phase1(n_edges, n_nodes, d):
    nchunk = n_edges // _CH
    ch_per_tile = nchunk // _NW
    ch_rem = nchunk % _NW
    n_groups = (ch_per_tile + 1 + _NBUF - 1) // _NBUF
    last_slab = n_nodes - (_NS - 1) * _SLAB
    mesh = plsc.VectorSubcoreMesh(core_axis_name="c", subcore_axis_name="s")

    @functools.partial(
        pl.kernel,
        mesh=mesh,
        out_type=(
            jax.ShapeDtypeStruct((_NC, n_nodes, d), jnp.float32),
            jax.ShapeDtypeStruct((_NC, n_nodes), jnp.float32),
        ),
        scratch_types=(
            [pltpu.VMEM((_CH,), jnp.int32) for _ in range(_NBUF)]
            + [pltpu.VMEM((_CH, d), jnp.float32) for _ in range(_NBUF)]
            + [pltpu.VMEM((_CH,), jnp.float32),
               pltpu.VMEM_SHARED((n_nodes, d), jnp.float32),
               pltpu.VMEM_SHARED((n_nodes,), jnp.float32)]
            + [pltpu.SemaphoreType.DMA for _ in range(2 * _NBUF)]
        ),
    )
    def phase1(energy, centers, sums_out, cnts_out, *refs):
        idx = refs[:_NBUF]
        rows = refs[_NBUF:2 * _NBUF]
        ones_v = refs[2 * _NBUF]
        acc_sh = refs[2 * _NBUF + 1]
        cnt_sh = refs[2 * _NBUF + 2]
        sems = refs[2 * _NBUF + 3:3 * _NBUF + 3]
        csems = refs[3 * _NBUF + 3:]

        cid = lax.axis_index("c")
        sid = lax.axis_index("s")
        wid = sid * _NC + cid
        slab = sid * _SLAB

        n_ch = ch_per_tile + jnp.where(wid < ch_rem, 1, 0)

        def issue_loads(o, b):
            base = (wid + o * _NW) * _CH
            pltpu.async_copy(centers.at[pl.ds(base, _CH)], idx[b], sems[b])
            pltpu.async_copy(energy.at[pl.ds(base, _CH)], rows[b], sems[b])

        def issue_idx(o, b):
            base = (wid + o * _NW) * _CH
            pltpu.async_copy(centers.at[pl.ds(base, _CH)], idx[b], sems[b])

        def issue_rows(o, b):
            base = (wid + o * _NW) * _CH
            pltpu.async_copy(energy.at[pl.ds(base, _CH)], rows[b], sems[b])

        @pl.when(0 < n_ch)
        def _():
            issue_idx(0, 0)

        for b in range(1, _NBUF):
            @pl.when(b < n_ch)
            def _():
                issue_loads(b, b)

        zv = jnp.zeros((16,), jnp.float32)

        def zrow(i, carry):
            for j in range(d // 16):
                rows[0][i, pl.ds(j * 16, 16)] = zv
            return carry

        lax.fori_loop(0, _CH, zrow, 0)

        def zero_slab(length):
            full = length // _CH
            tail = length - full * _CH
            for k in range(full):
                pltpu.sync_copy(rows[0],
                                acc_sh.at[pl.ds(slab + k * _CH, _CH)])
            if tail:
                pltpu.sync_copy(rows[0].at[pl.ds(0, tail)],
                                acc_sh.at[pl.ds(slab + full * _CH, tail)])
            cfull = length // d
            ctail = length - cfull * d
            for k in range(cfull):
                pltpu.sync_copy(rows[0].at[k],
                                cnt_sh.at[pl.ds(slab + k * d, d)])
            if ctail:
                pltpu.sync_copy(rows[0].at[cfull, pl.ds(0, ctail)],
                                cnt_sh.at[pl.ds(slab + cfull * d, ctail)])

        @pl.when(sid < _NS - 1)
        def _():
            zero_slab(_SLAB)

        @pl.when(sid == _NS - 1)
        def _():
            zero_slab(last_slab)

        for j in range(_CH // 16):
            ones_v[pl.ds(j * 16, 16)] = jnp.full((16,), 1.0, jnp.float32)

        @pl.when(0 < n_ch)
        def _():
            issue_rows(0, 0)

        plsc.subcore_barrier()

        def wait_loads(b):
            pltpu.make_async_copy(centers.at[pl.ds(0, _CH)], idx[b],
                                  sems[b]).wait()
            pltpu.make_async_copy(energy.at[pl.ds(0, _CH)], rows[b],
                                  sems[b]).wait()

        def scatter(b):
            pltpu.async_copy(ones_v, cnt_sh.at[idx[b]], csems[b], add=True)
            pltpu.sync_copy(rows[b], acc_sh.at[idx[b]], add=True)
            pltpu.make_async_copy(ones_v, cnt_sh.at[idx[b]],
                                  csems[b]).wait()

        def group_step(j, carry):
            for b in range(_NBUF):
                o = _NBUF * j + b

                @pl.when(o < n_ch)
                def _():
                    wait_loads(b)
                    scatter(b)

                    @pl.when(o + _NBUF < n_ch)
                    def _():
                        issue_loads(o + _NBUF, b)
            return carry

        lax.fori_loop(0, n_groups, group_step, 0)
        plsc.subcore_barrier()

        def write_slab(length):
            pltpu.sync_copy(
                acc_sh.at[pl.ds(slab, length)],
                sums_out.at[cid, pl.ds(slab, length)],
            )

        @pl.when(sid < _NS - 1)
        def _():
            write_slab(_SLAB)

        @pl.when(sid == _NS - 1)
        def _():
            write_slab(last_slab)

        @pl.when(sid == 0)
        def _():
            pltpu.sync_copy(cnt_sh, cnts_out.at[cid])

    return phase1


def _make_phase2(n_nodes, d, rblk):
    def body(f_ref, p_ref, c_ref, o_ref):
        s = p_ref[0] + p_ref[1]
        c = c_ref[0] + c_ref[1]
        o_ref[...] = (s / jnp.maximum(c, 1.0)) * f_ref[0]

    return pl.pallas_call(
        body,
        grid=(n_nodes // rblk,),
        in_specs=[
            pl.BlockSpec(memory_space=pltpu.SMEM),
            pl.BlockSpec((_NC, rblk, d), lambda i: (0, i, 0)),
            pl.BlockSpec((_NC, rblk, 1), lambda i: (0, i, 0)),
        ],
        out_specs=pl.BlockSpec((rblk, d), lambda i: (i, 0)),
        out_shape=jax.ShapeDtypeStruct((n_nodes, d), jnp.float32),
    )


def kernel(edge_energy, edge_index, atom_type, avg_num_neighbors):
    n_edges, d = edge_energy.shape
    n_nodes = atom_type.shape[0]
    if edge_index.dtype != jnp.int32:
        edge_index = edge_index.astype(jnp.int32)
    centers_flat = edge_index.reshape(-1)
    sums, cnts = _make_phase1(n_edges, n_nodes, d)(edge_energy, centers_flat)
    factor = (1.0 / jnp.sqrt(jnp.asarray(avg_num_neighbors, jnp.float32)))
    factor = factor.reshape(1)
    cnts3 = cnts.reshape(_NC, n_nodes, 1)
    return _make_phase2(n_nodes, d, 2000)(factor, sums, cnts3)


# speedup vs baseline: 1.0046x; 1.0046x over previous
import functools

import jax
import jax.numpy as jnp
from jax import lax
from jax.experimental import pallas as pl
from jax.experimental.pallas import tpu as pltpu
from jax.experimental.pallas import tpu_sc as plsc

_NC = 2
_NS = 16
_NW = _NC * _NS
_CH = 80
_NBUF = 4
_SLAB = 632


def _make_phase1(n_edges, n_nodes, d):
    nchunk = n_edges // _CH
    ch_per_tile = nchunk // _NW
    ch_rem = nchunk % _NW
    n_groups = (ch_per_tile + 1 + _NBUF - 1) // _NBUF
    last_slab = n_nodes - (_NS - 1) * _SLAB
    mesh = plsc.VectorSubcoreMesh(core_axis_name="c", subcore_axis_name="s")

    @functools.partial(
        pl.kernel,
        mesh=mesh,
        out_type=(
            jax.ShapeDtypeStruct((_NC, n_nodes, d), jnp.float32),
            jax.ShapeDtypeStruct((_NC, n_nodes), jnp.float32),
        ),
        scratch_types=(
            [pltpu.VMEM((_CH,), jnp.int32) for _ in range(_NBUF)]
            + [pltpu.VMEM((_CH, d), jnp.float32) for _ in range(_NBUF)]
            + [pltpu.VMEM((_CH,), jnp.float32),
               pltpu.VMEM_SHARED((n_nodes, d), jnp.float32),
               pltpu.VMEM_SHARED((n_nodes,), jnp.float32)]
            + [pltpu.SemaphoreType.DMA for _ in range(2 * _NBUF)]
        ),
    )
    def phase1(energy, centers, sums_out, cnts_out, *refs):
        idx = refs[:_NBUF]
        rows = refs[_NBUF:2 * _NBUF]
        ones_v = refs[2 * _NBUF]
        acc_sh = refs[2 * _NBUF + 1]
        cnt_sh = refs[2 * _NBUF + 2]
        sems = refs[2 * _NBUF + 3:3 * _NBUF + 3]
        csems = refs[3 * _NBUF + 3:]

        cid = lax.axis_index("c")
        sid = lax.axis_index("s")
        wid = sid * _NC + cid
        slab = sid * _SLAB

        n_ch = ch_per_tile + jnp.where(wid < ch_rem, 1, 0)

        def issue_loads(o, b):
            base = (wid + o * _NW) * _CH
            pltpu.async_copy(centers.at[pl.ds(base, _CH)], idx[b], sems[b])
            pltpu.async_copy(energy.at[pl.ds(base, _CH)], rows[b], sems[b])

        def issue_idx(o, b):
            base = (wid + o * _NW) * _CH
            pltpu.async_copy(centers.at[pl.ds(base, _CH)], idx[b], sems[b])

        def issue_rows(o, b):
            base = (wid + o * _NW) * _CH
            pltpu.async_copy(energy.at[pl.ds(base, _CH)], rows[b], sems[b])

        @pl.when(0 < n_ch)
        def _():
            issue_idx(0, 0)

        for b in range(1, _NBUF):
            @pl.when(b < n_ch)
            def _():
                issue_loads(b, b)

        zv = jnp.zeros((16,), jnp.float32)

        def zrow(i, carry):
            for j in range(d // 16):
                rows[0][i, pl.ds(j * 16, 16)] = zv
            return carry

        lax.fori_loop(0, _CH, zrow, 0)

        def zero_slab(length):
            full = length // _CH
            tail = length - full * _CH
            for k in range(full):
                pltpu.sync_copy(rows[0],
                                acc_sh.at[pl.ds(slab + k * _CH, _CH)])
            if tail:
                pltpu.sync_copy(rows[0].at[pl.ds(0, tail)],
                                acc_sh.at[pl.ds(slab + full * _CH, tail)])
            cfull = length // d
            ctail = length - cfull * d
            for k in range(cfull):
                pltpu.sync_copy(rows[0].at[k],
                                cnt_sh.at[pl.ds(slab + k * d, d)])
            if ctail:
                pltpu.sync_copy(rows[0].at[cfull, pl.ds(0, ctail)],
                                cnt_sh.at[pl.ds(slab + cfull * d, ctail)])

        @pl.when(sid < _NS - 1)
        def _():
            zero_slab(_SLAB)

        @pl.when(sid == _NS - 1)
        def _():
            zero_slab(last_slab)

        for j in range(_CH // 16):
            ones_v[pl.ds(j * 16, 16)] = jnp.full((16,), 1.0, jnp.float32)

        @pl.when(0 < n_ch)
        def _():
            issue_rows(0, 0)

        plsc.subcore_barrier()

        def wait_loads(b):
            pltpu.make_async_copy(centers.at[pl.ds(0, _CH)], idx[b],
                                  sems[b]).wait()
            pltpu.make_async_copy(energy.at[pl.ds(0, _CH)], rows[b],
                                  sems[b]).wait()

        def scatter(b):
            pltpu.async_copy(ones_v, cnt_sh.at[idx[b]], csems[b], add=True)
            pltpu.sync_copy(rows[b], acc_sh.at[idx[b]], add=True)
            pltpu.make_async_copy(ones_v, cnt_sh.at[idx[b]],
                                  csems[b]).wait()

        def group_step(j, carry):
            for b in range(_NBUF):
                o = _NBUF * j + b

                @pl.when(o < n_ch)
                def _():
                    wait_loads(b)
                    scatter(b)

                    @pl.when(o + _NBUF < n_ch)
                    def _():
                        issue_loads(o + _NBUF, b)
            return carry

        lax.fori_loop(0, n_groups, group_step, 0)
        plsc.subcore_barrier()

        def write_slab(length):
            pltpu.sync_copy(
                acc_sh.at[pl.ds(slab, length)],
                sums_out.at[cid, pl.ds(slab, length)],
            )

        @pl.when(sid < _NS - 1)
        def _():
            write_slab(_SLAB)

        @pl.when(sid == _NS - 1)
        def _():
            write_slab(last_slab)

        @pl.when(sid == 0)
        def _():
            pltpu.sync_copy(cnt_sh, cnts_out.at[cid])

    return phase1


def _make_phase2(n_nodes, d, rblk):
    def body(f_ref, p_ref, c_ref, o_ref):
        s = p_ref[0] + p_ref[1]
        c = c_ref[0] + c_ref[1]
        o_ref[...] = (s / jnp.maximum(c, 1.0)) * f_ref[0]

    return pl.pallas_call(
        body,
        grid=(n_nodes // rblk,),
        in_specs=[
            pl.BlockSpec(memory_space=pltpu.SMEM),
            pl.BlockSpec((_NC, rblk, d), lambda i: (0, i, 0)),
            pl.BlockSpec((_NC, rblk, 1), lambda i: (0, i, 0)),
        ],
        out_specs=pl.BlockSpec((rblk, d), lambda i: (i, 0)),
        out_shape=jax.ShapeDtypeStruct((n_nodes, d), jnp.float32),
    )


def kernel(edge_energy, edge_index, atom_type, avg_num_neighbors):
    n_edges, d = edge_energy.shape
    n_nodes = atom_type.shape[0]
    if edge_index.dtype != jnp.int32:
        edge_index = edge_index.astype(jnp.int32)
    centers_flat = edge_index.reshape(-1)
    sums, cnts = _make_phase1(n_edges, n_nodes, d)(edge_energy, centers_flat)
    factor = (1.0 / jnp.sqrt(jnp.asarray(avg_num_neighbors, jnp.float32)))
    factor = factor.reshape(1)
    cnts3 = cnts.reshape(_NC, n_nodes, 1)
    return _make_phase2(n_nodes, d, 2000)(factor, sums, cnts3)
